# baseline (speedup 1.0000x reference)
import functools

import jax
import jax.numpy as jnp
from jax import lax
from jax.experimental import pallas as pl
from jax.experimental.pallas import tpu as pltpu
from jax.experimental.pallas import tpu_sc as plsc

_D = 128
_NC = 2
_NS = 16
_NW = _NC * _NS
_K = 4


def _gather_kernel(num_ids, chunk):
    b_per_w = num_ids // _NW
    n_chunks = b_per_w // chunk
    mesh = plsc.VectorSubcoreMesh(core_axis_name="c", subcore_axis_name="s")

    @functools.partial(
        pl.kernel,
        mesh=mesh,
        out_type=jax.ShapeDtypeStruct((num_ids, _D), jnp.float32),
        scratch_types=[
            pltpu.VMEM((b_per_w,), jnp.int32),
            pltpu.VMEM((chunk, _D), jnp.float32),
            pltpu.VMEM((chunk, _D), jnp.float32),
            pltpu.SemaphoreType.DMA,
            pltpu.SemaphoreType.DMA,
            pltpu.SemaphoreType.DMA,
            pltpu.SemaphoreType.DMA,
        ],
    )
    def body(idx_hbm, table_hbm, out_hbm, idx_v, rows_a, rows_b,
             gsem_a, gsem_b, osem_a, osem_b):
        wid = lax.axis_index("s") * _NC + lax.axis_index("c")
        base = wid * b_per_w
        rows = (rows_a, rows_b)
        gsem = (gsem_a, gsem_b)
        osem = (osem_a, osem_b)

        pltpu.sync_copy(idx_hbm.at[pl.ds(base, b_per_w)], idx_v)

        def gather(c, s):
            return pltpu.async_copy(
                table_hbm.at[idx_v.at[pl.ds(c * chunk, chunk)]], rows[s],
                gsem[s])

        def writeback(c, s):
            return pltpu.async_copy(
                rows[s], out_hbm.at[pl.ds(base + c * chunk, chunk)], osem[s])

        pending_g = {0: gather(0, 0)}
        pending_o = {}
        for c in range(n_chunks):
            s = c % 2
            pending_g.pop(s).wait()
            if c + 1 < n_chunks:
                s2 = (c + 1) % 2
                if s2 in pending_o:
                    pending_o.pop(s2).wait()
                pending_g[s2] = gather(c + 1, s2)
            pending_o[s] = writeback(c, s)
        for o in pending_o.values():
            o.wait()

    return body


def kernel(token_ids, weight):
    b, s = token_ids.shape
    flat = token_ids.reshape(-1).astype(jnp.int32)
    part_ids = (b // _K) * s
    gk = _gather_kernel(part_ids, 400)
    parts = []
    for i in range(_K):
        rows = gk(lax.dynamic_slice(flat, (i * part_ids,), (part_ids,)), weight)
        parts.append(rows.reshape(b // _K, s, _D))
    return jnp.concatenate(parts, axis=0)


# speedup vs baseline: 2.0975x; 2.0975x over previous
import functools

import jax
import jax.numpy as jnp
from jax import lax
from jax.experimental import pallas as pl
from jax.experimental.pallas import tpu as pltpu
from jax.experimental.pallas import tpu_sc as plsc

_D = 128
_NC = 2
_NS = 16
_NW = _NC * _NS


def _gather_kernel(batch, seq, seq_pad, rows_per_chunk):
    rows_per_w = batch // _NW
    n_chunks = rows_per_w // rows_per_chunk
    ids_per_w = rows_per_w * seq_pad
    chunk_ids = rows_per_chunk * seq_pad
    mesh = plsc.VectorSubcoreMesh(core_axis_name="c", subcore_axis_name="s")

    @functools.partial(
        pl.kernel,
        mesh=mesh,
        out_type=jax.ShapeDtypeStruct((batch, seq, _D), jnp.float32),
        scratch_types=[
            pltpu.VMEM((ids_per_w,), jnp.int32),
            pltpu.VMEM((chunk_ids, _D), jnp.float32),
            pltpu.VMEM((chunk_ids, _D), jnp.float32),
            pltpu.SemaphoreType.DMA,
            pltpu.SemaphoreType.DMA,
            pltpu.SemaphoreType.DMA,
            pltpu.SemaphoreType.DMA,
        ],
        compiler_params=pltpu.CompilerParams(has_side_effects=True),
    )
    def body(idx_hbm, table_hbm, out_hbm, idx_v, rows_a, rows_b,
             gsem_a, gsem_b, osem_a, osem_b):
        wid = lax.axis_index("s") * _NC + lax.axis_index("c")
        row_base = wid * rows_per_w
        rows = (rows_a, rows_b)
        gsem = (gsem_a, gsem_b)
        osem = (osem_a, osem_b)

        pltpu.sync_copy(idx_hbm.at[pl.ds(wid * ids_per_w, ids_per_w)], idx_v)

        def gather(c, s):
            return pltpu.async_copy(
                table_hbm.at[idx_v.at[pl.ds(c * chunk_ids, chunk_ids)]],
                rows[s], gsem[s])

        def writeback(c, s):
            copies = []
            for j in range(rows_per_chunk):
                copies.append(pltpu.async_copy(
                    rows[s].at[pl.ds(j * seq_pad, seq)],
                    out_hbm.at[row_base + c * rows_per_chunk + j],
                    osem[s]))
            return copies

        pending_g = {0: gather(0, 0)}
        pending_o = {}
        for c in range(n_chunks):
            s = c % 2
            pending_g.pop(s).wait()
            if c + 1 < n_chunks:
                s2 = (c + 1) % 2
                for o in pending_o.pop(s2, ()):
                    o.wait()
                pending_g[s2] = gather(c + 1, s2)
            pending_o[s] = writeback(c, s)
        for os_ in pending_o.values():
            for o in os_:
                o.wait()

    return body


def kernel(token_ids, weight):
    b, s = token_ids.shape
    s_pad = 56
    ids32 = token_ids.astype(jnp.int32)
    ids = jnp.concatenate([ids32, ids32[:, : s_pad - s]], axis=1)
    flat = ids.reshape(-1)
    return _gather_kernel(b, s, s_pad, 8)(flat, weight)


# speedup vs baseline: 3.9430x; 1.8798x over previous
import functools

import jax
import jax.numpy as jnp
from jax import lax
from jax.experimental import pallas as pl
from jax.experimental.pallas import tpu as pltpu
from jax.experimental.pallas import tpu_sc as plsc

_D = 128
_NC = 2
_NS = 16
_NW = _NC * _NS


def _gather_kernel(num_ids, chunk):
    b_per_w = num_ids // _NW
    n_chunks = b_per_w // chunk
    mesh = plsc.VectorSubcoreMesh(core_axis_name="c", subcore_axis_name="s")

    @functools.partial(
        pl.kernel,
        mesh=mesh,
        out_type=jax.ShapeDtypeStruct((num_ids, _D), jnp.float32),
        scratch_types=[
            pltpu.VMEM((b_per_w,), jnp.int32),
            pltpu.VMEM((chunk, _D), jnp.float32),
            pltpu.VMEM((chunk, _D), jnp.float32),
            pltpu.SemaphoreType.DMA,
            pltpu.SemaphoreType.DMA,
            pltpu.SemaphoreType.DMA,
            pltpu.SemaphoreType.DMA,
        ],
    )
    def body(idx_hbm, table_hbm, out_hbm, idx_v, rows_a, rows_b,
             gsem_a, gsem_b, osem_a, osem_b):
        wid = lax.axis_index("s") * _NC + lax.axis_index("c")
        base = wid * b_per_w
        rows = (rows_a, rows_b)
        gsem = (gsem_a, gsem_b)
        osem = (osem_a, osem_b)

        pltpu.sync_copy(idx_hbm.at[pl.ds(base, b_per_w)], idx_v)

        def gather(c, s):
            return pltpu.async_copy(
                table_hbm.at[idx_v.at[pl.ds(c * chunk, chunk)]], rows[s],
                gsem[s])

        def writeback(c, s):
            return pltpu.async_copy(
                rows[s], out_hbm.at[pl.ds(base + c * chunk, chunk)], osem[s])

        pending_g = {0: gather(0, 0)}
        pending_o = {}
        for c in range(n_chunks):
            s = c % 2
            pending_g.pop(s).wait()
            if c + 1 < n_chunks:
                s2 = (c + 1) % 2
                if s2 in pending_o:
                    pending_o.pop(s2).wait()
                pending_g[s2] = gather(c + 1, s2)
            pending_o[s] = writeback(c, s)
        for o in pending_o.values():
            o.wait()

    return body


def kernel(token_ids, weight):
    b, s = token_ids.shape
    flat_t_major = token_ids.T.reshape(-1).astype(jnp.int32)
    out = _gather_kernel(b * s, 400)(flat_t_major, weight)
    return out.reshape(s, b, _D).transpose(1, 0, 2)
